# baseline (speedup 1.0000x reference)
import functools

import jax
import jax.numpy as jnp
from jax import lax
from jax.experimental import pallas as pl
from jax.experimental.pallas import tpu as pltpu
from jax.experimental.pallas import tpu_sc as plsc

N = 4096
K = 32
HID = 128
ROWS = 256
NW = 32
NPW = N // NW


NV = N // 128
M_ROUNDS = 4


def _tree_min(vals):
    vals = list(vals)
    while len(vals) > 1:
        nxt = [jnp.minimum(a, b) for a, b in zip(vals[::2], vals[1::2])]
        if len(vals) % 2:
            nxt.append(vals[-1])
        vals = nxt
    return vals[0]


def _full_extract(d, cols):
    inf = jnp.float32(jnp.inf)
    out = []
    for _ in range(K):
        m = jnp.min(d, axis=1, keepdims=True)
        idx = jnp.min(jnp.where(d <= m, cols, N), axis=1, keepdims=True)
        out.append(idx)
        d = jnp.where(cols == idx, inf, d)
    return jnp.concatenate(out, axis=1)


def _topk_body(sqc_ref, sqr_ref, xb_ref, xa_ref, w_ref, g_ref, p_ref, s_ref):
    in_dim = xb_ref.shape[1]
    xb = xb_ref[...]
    xy = lax.dot_general(xb, xa_ref[...], (((1,), (1,)), ((), ())),
                         preferred_element_type=jnp.float32)
    d2 = sqc_ref[...] + sqr_ref[...] - 2.0 * xy
    d0 = jnp.sqrt(jnp.maximum(d2, 0.0))
    inf = jnp.float32(jnp.inf)
    rows = d0.shape[0]

    sl = [d0[:, v * 128:(v + 1) * 128] for v in range(NV)]
    lane = lax.broadcasted_iota(jnp.int32, (rows, 128), 1)

    def _round(sl):
        cm = _tree_min(sl)
        eqs = [s == cm for s in sl]
        argv = _tree_min([jnp.where(eqs[v], v, NV) for v in range(NV)])
        col = argv * 128 + lane
        new_sl = [jnp.where(eqs[v] & (argv == v), inf, sl[v])
                  for v in range(NV)]
        return cm, col, new_sl

    cand_v, cand_c = [], []
    for _ in range(M_ROUNDS):
        cm, col, sl = _round(sl)
        cand_v.append(cm)
        cand_c.append(col)

    cm_r, col_r, _ = _round(sl)
    rmin = jnp.min(cm_r, axis=1, keepdims=True)
    rcol = jnp.min(jnp.where(cm_r <= rmin, col_r, N), axis=1, keepdims=True)

    cv = jnp.concatenate(cand_v, axis=1)
    cc = jnp.concatenate(cand_c, axis=1)
    idxs = []
    for _ in range(K):
        m = jnp.min(cv, axis=1, keepdims=True)
        sel = cv <= m
        c = jnp.min(jnp.where(sel, cc, N), axis=1, keepdims=True)
        idxs.append(c)
        cv = jnp.where(sel & (cc == c), inf, cv)
    g_cand = jnp.concatenate(idxs, axis=1)
    m32, c32 = m, c

    viol = (rmin < m32) | ((rmin == m32) & (rcol < c32))
    cols_full = lax.broadcasted_iota(jnp.int32, d0.shape, 1)
    g_ref[...] = lax.cond(
        jnp.any(viol),
        lambda: _full_extract(d0, cols_full),
        lambda: g_cand)

    w = w_ref[...]
    p_ref[...] = lax.dot_general(xb, w[:, :in_dim], (((1,), (1,)), ((), ())),
                                 preferred_element_type=jnp.float32)
    s_ref[...] = lax.dot_general(xb, w[:, in_dim:], (((1,), (1,)), ((), ())),
                                 preferred_element_type=jnp.float32)


def _topk_ps(x, w):
    in_dim = x.shape[1]
    sq = jnp.sum(x * x, axis=1)
    return pl.pallas_call(
        _topk_body,
        grid=(N // ROWS,),
        in_specs=[
            pl.BlockSpec((ROWS, 1), lambda i: (i, 0)),
            pl.BlockSpec((1, N), lambda i: (0, 0)),
            pl.BlockSpec((ROWS, in_dim), lambda i: (i, 0)),
            pl.BlockSpec((N, in_dim), lambda i: (0, 0)),
            pl.BlockSpec((HID, 2 * in_dim), lambda i: (0, 0)),
        ],
        out_specs=[
            pl.BlockSpec((ROWS, K), lambda i: (i, 0)),
            pl.BlockSpec((ROWS, HID), lambda i: (i, 0)),
            pl.BlockSpec((ROWS, HID), lambda i: (i, 0)),
        ],
        out_shape=[
            jax.ShapeDtypeStruct((N, K), jnp.int32),
            jax.ShapeDtypeStruct((N, HID), jnp.float32),
            jax.ShapeDtypeStruct((N, HID), jnp.float32),
        ],
    )(sq[:, None], sq[None, :], x, x, w)


GRP = 1
NGRP = NPW // GRP
GK = GRP * K
NBUF = 4


@functools.cache
def _make_agg():
    mesh = plsc.VectorSubcoreMesh(core_axis_name="c", subcore_axis_name="s")
    return pl.kernel(
        _agg_body,
        out_type=jax.ShapeDtypeStruct((N * HID,), jnp.float32),
        mesh=mesh,
        scratch_types=[
            pltpu.VMEM((NPW * K,), jnp.int32),
            pltpu.VMEM((NPW * HID,), jnp.float32),
            pltpu.VMEM((NPW * HID,), jnp.float32),
            [pltpu.VMEM((GK, HID), jnp.float32)] * NBUF,
            [pltpu.SemaphoreType.DMA] * NBUF,
        ],
    )


def _agg_body(p_hbm, s_hbm, g_hbm, out_hbm, idx_v, s_v, o_v, rings, sems):
    wid = lax.axis_index("s") * 2 + lax.axis_index("c")
    base = wid * NPW
    pltpu.sync_copy(g_hbm.at[pl.ds(base * K, NPW * K)], idx_v)
    pltpu.sync_copy(s_hbm.at[pl.ds(base * HID, NPW * HID)], s_v)

    def issue(g, b):
        pltpu.async_copy(p_hbm.at[idx_v.at[pl.ds(g * GK, GK)]],
                         rings[b], sems[b])

    def wait(b):
        pltpu.make_async_copy(p_hbm.at[pl.ds(0, GK)], rings[b], sems[b]).wait()

    def compute(g, b):
        buf = rings[b]
        for u in range(GRP):
            for c in range(HID // 16):
                sl = pl.ds((g * GRP + u) * HID + c * 16, 16)
                s_c = s_v[sl]
                csl = pl.ds(c * 16, 16)
                acc0 = jnp.clip(buf[u * K, csl] + s_c, -1.0, 1.0)
                acc1 = jnp.clip(buf[u * K + 1, csl] + s_c, -1.0, 1.0)
                for j in range(2, K, 2):
                    acc0 = acc0 + jnp.clip(buf[u * K + j, csl] + s_c,
                                           -1.0, 1.0)
                    acc1 = acc1 + jnp.clip(buf[u * K + j + 1, csl] + s_c,
                                           -1.0, 1.0)
                o_v[sl] = (acc0 + acc1) * (1.0 / K)

    for b in range(NBUF):
        issue(b, b)

    def body(t, _):
        g = t * NBUF
        for b in range(NBUF):
            wait(b)
            compute(g + b, b)
            issue(jnp.minimum(g + b + NBUF, NGRP - 1), b)
        return 0

    lax.fori_loop(0, NGRP // NBUF, body, 0)
    for b in range(NBUF):
        wait(b)
    pltpu.sync_copy(o_v, out_hbm.at[pl.ds(base * HID, NPW * HID)])


def _proj_body(m_ref, x_ref, w2_ref, o_ref):
    cat = jnp.concatenate([m_ref[...], x_ref[...]], axis=-1)
    o_ref[...] = lax.dot_general(cat, w2_ref[...], (((1,), (1,)), ((), ())),
                                 preferred_element_type=jnp.float32)


def _proj(m, x, w2):
    return pl.pallas_call(
        _proj_body,
        out_shape=jax.ShapeDtypeStruct((N, w2.shape[0]), jnp.float32),
    )(m, x, w2)


def _layer(x, w, w2):
    graph, p, s = _topk_ps(x, w)
    m = _make_agg()(p, s.reshape(-1), graph.reshape(-1))
    return _proj(m.reshape(N, HID), x, w2)


def kernel(x, w_a, w2_a, w_b, w2_b):
    h1 = _layer(x, w_a, w2_a)
    h2 = _layer(h1, w_b, w2_b)
    return h2


# speedup vs baseline: 1.0193x; 1.0193x over previous
import functools

import jax
import jax.numpy as jnp
from jax import lax
from jax.experimental import pallas as pl
from jax.experimental.pallas import tpu as pltpu
from jax.experimental.pallas import tpu_sc as plsc

N = 4096
K = 32
HID = 128
ROWS = 128
NW = 32
NPW = N // NW


NV = N // 128
M_ROUNDS = 5


def _tree_min(vals):
    vals = list(vals)
    while len(vals) > 1:
        nxt = [jnp.minimum(a, b) for a, b in zip(vals[::2], vals[1::2])]
        if len(vals) % 2:
            nxt.append(vals[-1])
        vals = nxt
    return vals[0]


def _full_extract(d, cols):
    inf = jnp.float32(jnp.inf)
    out = []
    for _ in range(K):
        m = jnp.min(d, axis=1, keepdims=True)
        idx = jnp.min(jnp.where(d <= m, cols, N), axis=1, keepdims=True)
        out.append(idx)
        d = jnp.where(cols == idx, inf, d)
    return jnp.concatenate(out, axis=1)


def _topk_body(sqc_ref, sqr_ref, xb_ref, xa_ref, w_ref, g_ref, p_ref, s_ref):
    in_dim = xb_ref.shape[1]
    xb = xb_ref[...]
    xy = lax.dot_general(xb, xa_ref[...], (((1,), (1,)), ((), ())),
                         preferred_element_type=jnp.float32)
    d2 = sqc_ref[...] + sqr_ref[...] - 2.0 * xy
    d0 = jnp.sqrt(jnp.maximum(d2, 0.0))
    inf = jnp.float32(jnp.inf)
    rows = d0.shape[0]

    sl = [d0[:, v * 128:(v + 1) * 128] for v in range(NV)]
    lane = lax.broadcasted_iota(jnp.int32, (rows, 128), 1)

    def _round(sl):
        cm = _tree_min(sl)
        eqs = [s == cm for s in sl]
        argv = _tree_min([jnp.where(eqs[v], v, NV) for v in range(NV)])
        col = argv * 128 + lane
        new_sl = [jnp.where(eqs[v] & (argv == v), inf, sl[v])
                  for v in range(NV)]
        return cm, col, new_sl

    cand_v, cand_c = [], []
    for _ in range(M_ROUNDS):
        cm, col, sl = _round(sl)
        cand_v.append(cm)
        cand_c.append(col)

    cm_r, col_r, _ = _round(sl)
    rmin = jnp.min(cm_r, axis=1, keepdims=True)
    rcol = jnp.min(jnp.where(cm_r <= rmin, col_r, N), axis=1, keepdims=True)

    cv = jnp.concatenate(cand_v, axis=1)
    cc = jnp.concatenate(cand_c, axis=1)
    idxs = []
    for _ in range(K):
        m = jnp.min(cv, axis=1, keepdims=True)
        sel = cv <= m
        c = jnp.min(jnp.where(sel, cc, N), axis=1, keepdims=True)
        idxs.append(c)
        cv = jnp.where(sel & (cc == c), inf, cv)
    g_cand = jnp.concatenate(idxs, axis=1)
    m32, c32 = m, c

    viol = (rmin < m32) | ((rmin == m32) & (rcol < c32))
    cols_full = lax.broadcasted_iota(jnp.int32, d0.shape, 1)
    g_ref[...] = lax.cond(
        jnp.any(viol),
        lambda: _full_extract(d0, cols_full),
        lambda: g_cand)

    w = w_ref[...]
    p_ref[...] = lax.dot_general(xb, w[:, :in_dim], (((1,), (1,)), ((), ())),
                                 preferred_element_type=jnp.float32)
    s_ref[...] = lax.dot_general(xb, w[:, in_dim:], (((1,), (1,)), ((), ())),
                                 preferred_element_type=jnp.float32)


def _topk_ps(x, w):
    in_dim = x.shape[1]
    sq = jnp.sum(x * x, axis=1)
    return pl.pallas_call(
        _topk_body,
        grid=(N // ROWS,),
        in_specs=[
            pl.BlockSpec((ROWS, 1), lambda i: (i, 0)),
            pl.BlockSpec((1, N), lambda i: (0, 0)),
            pl.BlockSpec((ROWS, in_dim), lambda i: (i, 0)),
            pl.BlockSpec((N, in_dim), lambda i: (0, 0)),
            pl.BlockSpec((HID, 2 * in_dim), lambda i: (0, 0)),
        ],
        out_specs=[
            pl.BlockSpec((ROWS, K), lambda i: (i, 0)),
            pl.BlockSpec((ROWS, HID), lambda i: (i, 0)),
            pl.BlockSpec((ROWS, HID), lambda i: (i, 0)),
        ],
        out_shape=[
            jax.ShapeDtypeStruct((N, K), jnp.int32),
            jax.ShapeDtypeStruct((N, HID), jnp.float32),
            jax.ShapeDtypeStruct((N, HID), jnp.float32),
        ],
    )(sq[:, None], sq[None, :], x, x, w)


GRP = 1
NGRP = NPW // GRP
GK = GRP * K
NBUF = 4


@functools.cache
def _make_agg():
    mesh = plsc.VectorSubcoreMesh(core_axis_name="c", subcore_axis_name="s")
    return pl.kernel(
        _agg_body,
        out_type=jax.ShapeDtypeStruct((N * HID,), jnp.float32),
        mesh=mesh,
        scratch_types=[
            pltpu.VMEM((NPW * K,), jnp.int32),
            pltpu.VMEM((NPW * HID,), jnp.float32),
            pltpu.VMEM((NPW * HID,), jnp.float32),
            [pltpu.VMEM((GK, HID), jnp.float32)] * NBUF,
            [pltpu.SemaphoreType.DMA] * NBUF,
        ],
    )


def _agg_body(p_hbm, s_hbm, g_hbm, out_hbm, idx_v, s_v, o_v, rings, sems):
    wid = lax.axis_index("s") * 2 + lax.axis_index("c")
    base = wid * NPW
    pltpu.sync_copy(g_hbm.at[pl.ds(base * K, NPW * K)], idx_v)
    pltpu.sync_copy(s_hbm.at[pl.ds(base * HID, NPW * HID)], s_v)

    def issue(g, b):
        pltpu.async_copy(p_hbm.at[idx_v.at[pl.ds(g * GK, GK)]],
                         rings[b], sems[b])

    def wait(b):
        pltpu.make_async_copy(p_hbm.at[pl.ds(0, GK)], rings[b], sems[b]).wait()

    def compute(g, b):
        buf = rings[b]
        for u in range(GRP):
            for c in range(HID // 16):
                sl = pl.ds((g * GRP + u) * HID + c * 16, 16)
                s_c = s_v[sl]
                csl = pl.ds(c * 16, 16)
                acc0 = jnp.clip(buf[u * K, csl] + s_c, -1.0, 1.0)
                acc1 = jnp.clip(buf[u * K + 1, csl] + s_c, -1.0, 1.0)
                for j in range(2, K, 2):
                    acc0 = acc0 + jnp.clip(buf[u * K + j, csl] + s_c,
                                           -1.0, 1.0)
                    acc1 = acc1 + jnp.clip(buf[u * K + j + 1, csl] + s_c,
                                           -1.0, 1.0)
                o_v[sl] = (acc0 + acc1) * (1.0 / K)

    for b in range(NBUF):
        issue(b, b)

    def body(t, _):
        g = t * NBUF
        for b in range(NBUF):
            wait(b)
            compute(g + b, b)
            issue(jnp.minimum(g + b + NBUF, NGRP - 1), b)
        return 0

    lax.fori_loop(0, NGRP // NBUF, body, 0)
    for b in range(NBUF):
        wait(b)
    pltpu.sync_copy(o_v, out_hbm.at[pl.ds(base * HID, NPW * HID)])


def _proj_body(m_ref, x_ref, w2_ref, o_ref):
    cat = jnp.concatenate([m_ref[...], x_ref[...]], axis=-1)
    o_ref[...] = lax.dot_general(cat, w2_ref[...], (((1,), (1,)), ((), ())),
                                 preferred_element_type=jnp.float32)


def _proj(m, x, w2):
    return pl.pallas_call(
        _proj_body,
        out_shape=jax.ShapeDtypeStruct((N, w2.shape[0]), jnp.float32),
    )(m, x, w2)


def _layer(x, w, w2):
    graph, p, s = _topk_ps(x, w)
    m = _make_agg()(p, s.reshape(-1), graph.reshape(-1))
    return _proj(m.reshape(N, HID), x, w2)


def kernel(x, w_a, w2_a, w_b, w2_b):
    h1 = _layer(x, w_a, w2_a)
    h2 = _layer(h1, w_b, w2_b)
    return h2


# speedup vs baseline: 1.2727x; 1.2485x over previous
import functools

import jax
import jax.numpy as jnp
from jax import lax
from jax.experimental import pallas as pl
from jax.experimental.pallas import tpu as pltpu
from jax.experimental.pallas import tpu_sc as plsc

N = 4096
K = 32
HID = 128
ROWS = 256
NW = 32
NPW = N // NW


NV = N // 128
M_ROUNDS = 5


def _tree_min(vals):
    vals = list(vals)
    while len(vals) > 1:
        nxt = [jnp.minimum(a, b) for a, b in zip(vals[::2], vals[1::2])]
        if len(vals) % 2:
            nxt.append(vals[-1])
        vals = nxt
    return vals[0]


def _full_extract(d, cols):
    inf = jnp.float32(jnp.inf)
    out = []
    for _ in range(K):
        m = jnp.min(d, axis=1, keepdims=True)
        idx = jnp.min(jnp.where(d <= m, cols, N), axis=1, keepdims=True)
        out.append(idx)
        d = jnp.where(cols == idx, inf, d)
    return jnp.concatenate(out, axis=1)


def _topk_body(sqc_ref, sqr_ref, xb_ref, xa_ref, w_ref, g_ref, p_ref, s_ref):
    in_dim = xb_ref.shape[1]
    xb = xb_ref[...]
    xy = lax.dot_general(xb, xa_ref[...], (((1,), (1,)), ((), ())),
                         preferred_element_type=jnp.float32)
    d2 = sqc_ref[...] + sqr_ref[...] - 2.0 * xy
    d0 = jnp.sqrt(jnp.maximum(d2, 0.0))
    inf = jnp.float32(jnp.inf)
    rows = d0.shape[0]

    sl = [d0[:, v * 128:(v + 1) * 128] for v in range(NV)]
    lane = lax.broadcasted_iota(jnp.int32, (rows, 128), 1)

    def _round(sl):
        cm = _tree_min(sl)
        eqs = [s == cm for s in sl]
        argv = _tree_min([jnp.where(eqs[v], v, NV) for v in range(NV)])
        col = argv * 128 + lane
        new_sl = [jnp.where(eqs[v] & (argv == v), inf, sl[v])
                  for v in range(NV)]
        return cm, col, new_sl

    cand_v, cand_c = [], []
    for _ in range(M_ROUNDS):
        cm, col, sl = _round(sl)
        cand_v.append(cm)
        cand_c.append(col)

    cm_r, col_r, _ = _round(sl)
    rmin = jnp.min(cm_r, axis=1, keepdims=True)
    rcol = jnp.min(jnp.where(cm_r <= rmin, col_r, N), axis=1, keepdims=True)

    cv = jnp.concatenate(cand_v, axis=1)
    cc = jnp.concatenate(cand_c, axis=1)
    idxs = []
    for _ in range(K):
        m = jnp.min(cv, axis=1, keepdims=True)
        sel = cv <= m
        c = jnp.min(jnp.where(sel, cc, N), axis=1, keepdims=True)
        idxs.append(c)
        cv = jnp.where(sel & (cc == c), inf, cv)
    g_cand = jnp.concatenate(idxs, axis=1)
    m32, c32 = m, c

    viol = (rmin < m32) | ((rmin == m32) & (rcol < c32))
    cols_full = lax.broadcasted_iota(jnp.int32, d0.shape, 1)
    g_ref[...] = lax.cond(
        jnp.any(viol),
        lambda: _full_extract(d0, cols_full),
        lambda: g_cand)

    w = w_ref[...]
    p_ref[...] = lax.dot_general(xb, w[:, :in_dim], (((1,), (1,)), ((), ())),
                                 preferred_element_type=jnp.float32)
    s_ref[...] = lax.dot_general(xb, w[:, in_dim:], (((1,), (1,)), ((), ())),
                                 preferred_element_type=jnp.float32)


def _topk_ps(x, w):
    in_dim = x.shape[1]
    sq = jnp.sum(x * x, axis=1)
    return pl.pallas_call(
        _topk_body,
        grid=(N // ROWS,),
        in_specs=[
            pl.BlockSpec((ROWS, 1), lambda i: (i, 0)),
            pl.BlockSpec((1, N), lambda i: (0, 0)),
            pl.BlockSpec((ROWS, in_dim), lambda i: (i, 0)),
            pl.BlockSpec((N, in_dim), lambda i: (0, 0)),
            pl.BlockSpec((HID, 2 * in_dim), lambda i: (0, 0)),
        ],
        out_specs=[
            pl.BlockSpec((ROWS, K), lambda i: (i, 0)),
            pl.BlockSpec((ROWS, HID), lambda i: (i, 0)),
            pl.BlockSpec((ROWS, HID), lambda i: (i, 0)),
        ],
        out_shape=[
            jax.ShapeDtypeStruct((N, K), jnp.int32),
            jax.ShapeDtypeStruct((N, HID), jnp.float32),
            jax.ShapeDtypeStruct((N, HID), jnp.float32),
        ],
    )(sq[:, None], sq[None, :], x, x, w)


GRP = 1
NGRP = NPW // GRP
GK = GRP * K
NBUF = 4


@functools.cache
def _make_agg():
    mesh = plsc.VectorSubcoreMesh(core_axis_name="c", subcore_axis_name="s")
    return pl.kernel(
        _agg_body,
        out_type=jax.ShapeDtypeStruct((N * HID,), jnp.float32),
        mesh=mesh,
        scratch_types=[
            pltpu.VMEM((NPW * K,), jnp.int32),
            pltpu.VMEM((NPW * HID,), jnp.float32),
            pltpu.VMEM((NPW * HID,), jnp.float32),
            [pltpu.VMEM((GK, HID), jnp.float32)] * NBUF,
            [pltpu.SemaphoreType.DMA] * NBUF,
            pltpu.SemaphoreType.DMA,
        ],
    )


def _agg_body(p_hbm, s_hbm, g_hbm, out_hbm, idx_v, s_v, o_v, rings, sems,
              sem_s):
    wid = lax.axis_index("s") * 2 + lax.axis_index("c")
    base = wid * NPW
    pltpu.sync_copy(g_hbm.at[pl.ds(base * K, NPW * K)], idx_v)
    s_copy = pltpu.async_copy(s_hbm.at[pl.ds(base * HID, NPW * HID)], s_v,
                              sem_s)

    def issue(g, b):
        pltpu.async_copy(p_hbm.at[idx_v.at[pl.ds(g * GK, GK)]],
                         rings[b], sems[b])

    def wait(b):
        pltpu.make_async_copy(p_hbm.at[pl.ds(0, GK)], rings[b], sems[b]).wait()

    def compute(g, b):
        buf = rings[b]
        for u in range(GRP):
            for c in range(HID // 16):
                sl = pl.ds((g * GRP + u) * HID + c * 16, 16)
                s_c = s_v[sl]
                csl = pl.ds(c * 16, 16)
                acc0 = jnp.clip(buf[u * K, csl] + s_c, -1.0, 1.0)
                acc1 = jnp.clip(buf[u * K + 1, csl] + s_c, -1.0, 1.0)
                for j in range(2, K, 2):
                    acc0 = acc0 + jnp.clip(buf[u * K + j, csl] + s_c,
                                           -1.0, 1.0)
                    acc1 = acc1 + jnp.clip(buf[u * K + j + 1, csl] + s_c,
                                           -1.0, 1.0)
                o_v[sl] = (acc0 + acc1) * (1.0 / K)

    for b in range(NBUF):
        issue(b, b)
    s_copy.wait()

    def body(t, _):
        g = t * NBUF
        for b in range(NBUF):
            wait(b)
            compute(g + b, b)
            issue(jnp.minimum(g + b + NBUF, NGRP - 1), b)
        return 0

    lax.fori_loop(0, NGRP // NBUF, body, 0)
    for b in range(NBUF):
        wait(b)
    pltpu.sync_copy(o_v, out_hbm.at[pl.ds(base * HID, NPW * HID)])


def _proj_body(m_ref, x_ref, w2_ref, o_ref):
    cat = jnp.concatenate([m_ref[...], x_ref[...]], axis=-1)
    o_ref[...] = lax.dot_general(cat, w2_ref[...], (((1,), (1,)), ((), ())),
                                 preferred_element_type=jnp.float32)


def _proj(m, x, w2):
    return pl.pallas_call(
        _proj_body,
        out_shape=jax.ShapeDtypeStruct((N, w2.shape[0]), jnp.float32),
    )(m, x, w2)


def _layer(x, w, w2):
    graph, p, s = _topk_ps(x, w)
    m = _make_agg()(p, s.reshape(-1), graph.reshape(-1))
    return _proj(m.reshape(N, HID), x, w2)


def kernel(x, w_a, w2_a, w_b, w2_b):
    h1 = _layer(x, w_a, w2_a)
    h2 = _layer(h1, w_b, w2_b)
    return h2
